# baseline (speedup 1.0000x reference)
import jax
import jax.numpy as jnp
from jax import lax
from jax.experimental import pallas as pl
from jax.experimental.pallas import tpu as pltpu
from jax.experimental.pallas import tpu_sc as plsc

N = 10000
DIN = 256
DH = 128
DHW = DH // 2
E = 160000
NSUB = 16
NCORE = 2
CHUNK = 128
NCHUNK = 80
IB = 8
NBLK = NCHUNK // IB
EPAD = NSUB * NCHUNK * CHUNK
NPAD = 10112
RPS = NPAD // NSUB
CW = 16


def _sc_body(x0i, x1i, x0f, x1f, srcs, dsts, zc, ones_h, agg_out, cnt_out,
             agg_sh, cnt_sh, sblk, dblk, gb0, gb1, fb, ones_v,
             sg0, sg1, ss):
    c = lax.axis_index("c")
    s = lax.axis_index("s")
    rows = pl.ds(s * RPS, RPS)

    pltpu.sync_copy(zc.at[rows], cnt_sh.at[rows])
    pltpu.sync_copy(ones_h, ones_v)

    @pl.when(c == 0)
    def _():
        pltpu.sync_copy(x0f.at[rows], agg_sh.at[rows])

    @pl.when(c == 1)
    def _():
        pltpu.sync_copy(x1f.at[rows], agg_sh.at[rows])

    plsc.subcore_barrier()

    def run(table, count_parity):
        def fire_gather(gb, r, sem):
            pltpu.async_copy(table.at[sblk.at[r]], gb, sem)

        def drain_gather(gb, sem):
            pltpu.make_async_copy(table.at[sblk.at[0]], gb, sem).wait()

        def convert(gb):
            def row(k, carry):
                for q in range(4):
                    w = gb[k, pl.ds(16 * q, 16)]
                    lo = plsc.bitcast(w << 16, jnp.float32)
                    hi = plsc.bitcast(w & jnp.int32(-65536), jnp.float32)
                    fb[k, pl.ds(32 * q, 16)] = lo
                    fb[k, pl.ds(32 * q + 16, 16)] = hi
                return carry
            lax.fori_loop(0, CHUNK, row, 0)

        def fire_scatter(r, counted):
            pltpu.async_copy(fb, agg_sh.at[dblk.at[r]], ss, add=True)
            if counted:
                pltpu.async_copy(ones_v, cnt_sh.at[dblk.at[r]], ss,
                                 add=True)

        def drain_scatter(counted):
            pltpu.make_async_copy(fb, agg_sh.at[dblk.at[0]], ss).wait()
            if counted:
                pltpu.make_async_copy(
                    ones_v, cnt_sh.at[dblk.at[0]], ss).wait()

        def load_blk(b):
            pltpu.sync_copy(srcs.at[s, pl.ds(b * IB, IB)], sblk)
            pltpu.sync_copy(dsts.at[s, pl.ds(b * IB, IB)], dblk)

        load_blk(0)
        fire_gather(gb0, 0, sg0)
        fire_gather(gb1, 1, sg1)

        def blk_body(b, carry):
            for r in range(IB):
                gb, sg = (gb0, sg0) if r % 2 == 0 else (gb1, sg1)
                counted = (r % 2) == count_parity
                prev_counted = ((r - 1) % 2) == count_parity
                drain_gather(gb, sg)
                if r > 0:
                    drain_scatter(prev_counted)
                convert(gb)
                fire_scatter(r, counted)
                if r < IB - 2:
                    fire_gather(gb, r + 2, sg)
            drain_scatter(((IB - 1) % 2) == count_parity)

            @pl.when(b < NBLK - 1)
            def _():
                load_blk(b + 1)
                fire_gather(gb0, 0, sg0)
                fire_gather(gb1, 1, sg1)
            return carry

        lax.fori_loop(0, NBLK, blk_body, 0)

    @pl.when(c == 0)
    def _():
        run(x0i, 0)

    @pl.when(c == 1)
    def _():
        run(x1i, 1)

    plsc.subcore_barrier()

    pltpu.sync_copy(agg_sh.at[rows], agg_out.at[c, rows])
    pltpu.sync_copy(cnt_sh.at[rows], cnt_out.at[c, rows])


def _sc_aggregate(x0i, x1i, x0f, x1f, srcs, dsts, zc, ones_h):
    mesh = plsc.VectorSubcoreMesh(core_axis_name="c", subcore_axis_name="s",
                                  num_cores=NCORE, num_subcores=NSUB)
    return pl.kernel(
        _sc_body,
        out_type=(jax.ShapeDtypeStruct((NCORE, NPAD, DH), jnp.float32),
                  jax.ShapeDtypeStruct((NCORE, NPAD, CW), jnp.float32)),
        mesh=mesh,
        scratch_types=[
            pltpu.VMEM_SHARED((NPAD, DH), jnp.float32),
            pltpu.VMEM_SHARED((NPAD, CW), jnp.float32),
            pltpu.VMEM((IB, CHUNK), jnp.int32),
            pltpu.VMEM((IB, CHUNK), jnp.int32),
            pltpu.VMEM((CHUNK, DHW), jnp.int32),
            pltpu.VMEM((CHUNK, DHW), jnp.int32),
            pltpu.VMEM((CHUNK, DH), jnp.float32),
            pltpu.VMEM((CHUNK, CW), jnp.float32),
            pltpu.SemaphoreType.DMA,
            pltpu.SemaphoreType.DMA,
            pltpu.SemaphoreType.DMA,
        ],
        compiler_params=pltpu.CompilerParams(use_tc_tiling_on_sc=False,
                                             needs_layout_passes=False),
    )(x0i, x1i, x0f, x1f, srcs, dsts, zc, ones_h)


def _tc_body(agg_ref, cnt_ref, w_ref, b_ref, out_ref):
    a = jnp.concatenate([agg_ref[0], agg_ref[1]], axis=1)
    cnt = cnt_ref[0, :, 0:1] + cnt_ref[1, :, 0:1]
    denom = 1.0 + jnp.maximum(cnt, 1.0)
    a = a / denom
    o = lax.dot_general(a, w_ref[:], (((1,), (1,)), ((), ())),
                        preferred_element_type=jnp.float32)
    o = o + b_ref[:]
    out_ref[:] = jnp.where(o >= 0.0, o, 0.2 * o)


def _tc_tail(agg, cnt, w, b2):
    bm = 512
    return pl.pallas_call(
        _tc_body,
        grid=(pl.cdiv(N, bm),),
        in_specs=[
            pl.BlockSpec((NCORE, bm, DH), lambda i: (0, i, 0)),
            pl.BlockSpec((NCORE, bm, CW), lambda i: (0, i, 0)),
            pl.BlockSpec((DIN, DIN), lambda i: (0, 0)),
            pl.BlockSpec((1, DIN), lambda i: (0, 0)),
        ],
        out_specs=pl.BlockSpec((bm, DIN), lambda i: (i, 0)),
        out_shape=jax.ShapeDtypeStruct((N, DIN), jnp.float32),
    )(agg, cnt, w, b2)


def _pack_half(xh):
    n = xh.shape[0]
    perm = xh.reshape(n, 4, 2, 16).transpose(0, 1, 3, 2)
    bf = perm.astype(jnp.bfloat16).reshape(n, DHW, 2)
    return lax.bitcast_convert_type(bf, jnp.int32)


def kernel(x, edge_index, W, b):
    src = edge_index[0].astype(jnp.int32)
    dst = edge_index[1].astype(jnp.int32)
    pad = EPAD - E
    src = jnp.concatenate([src, jnp.zeros((pad,), jnp.int32)])
    dst = jnp.concatenate(
        [dst, N + (jnp.arange(pad, dtype=jnp.int32) % (NPAD - N))])
    srcs = src.reshape(NSUB, NCHUNK, CHUNK)
    dsts = dst.reshape(NSUB, NCHUNK, CHUNK)
    xp = jnp.pad(x, ((0, NPAD - N), (0, 0)))
    x0f = xp[:, :DH]
    x1f = xp[:, DH:]
    x0i = _pack_half(x0f)
    x1i = _pack_half(x1f)
    zc = jnp.zeros((NPAD, CW), jnp.float32)
    ones_h = jnp.ones((CHUNK, CW), jnp.float32)
    agg, cnt = _sc_aggregate(x0i, x1i, x0f, x1f, srcs, dsts, zc, ones_h)
    return _tc_tail(agg, cnt, W, b.reshape(1, DIN))


# speedup vs baseline: 2.5826x; 2.5826x over previous
import jax
import jax.numpy as jnp
from jax import lax
from jax.experimental import pallas as pl
from jax.experimental.pallas import tpu as pltpu
from jax.experimental.pallas import tpu_sc as plsc

N = 10000
DIN = 256
DH = 128
E = 160000
NSUB = 16
NCORE = 2
CHUNK = 128
NCHUNK = 84
NQUAD = NCHUNK // 4
EPAD = NSUB * NCHUNK * CHUNK
NPAD = 10112
RPS = NPAD // NSUB
CW = 16


def _sc_body(x0b, x1b, srcs, dsts, zc, ones_h, agg_out, cnt_out,
             agg_sh, cnt_sh, src_v, dst_v, g0, g1, g2, g3, ones_v,
             sg0, sg1, sg2, sg3, ss0, ss1, ss2, ss3):
    c = lax.axis_index("c")
    s = lax.axis_index("s")
    rows = pl.ds(s * RPS, RPS)
    gbs = (g0, g1, g2, g3)
    sgs = (sg0, sg1, sg2, sg3)
    sss = (ss0, ss1, ss2, ss3)

    pltpu.sync_copy(srcs.at[s], src_v)
    pltpu.sync_copy(dsts.at[s], dst_v)
    pltpu.sync_copy(zc.at[rows], cnt_sh.at[rows])
    pltpu.sync_copy(ones_h, ones_v)

    @pl.when(c == 0)
    def _():
        pltpu.sync_copy(x0b.at[rows], agg_sh.at[rows])

    @pl.when(c == 1)
    def _():
        pltpu.sync_copy(x1b.at[rows], agg_sh.at[rows])

    plsc.subcore_barrier()

    def run(table, count_parity):
        def fire_gather(j, u):
            pltpu.async_copy(table.at[src_v.at[j]], gbs[u], sgs[u])

        def drain_gather(u):
            pltpu.make_async_copy(table.at[src_v.at[0]], gbs[u],
                                  sgs[u]).wait()

        def fire_scatter(j, u, counted):
            pltpu.async_copy(gbs[u], agg_sh.at[dst_v.at[j]], sss[u],
                             add=True)
            if counted:
                pltpu.async_copy(ones_v, cnt_sh.at[dst_v.at[j]], sss[u],
                                 add=True)

        def drain_scatter(u, counted):
            pltpu.make_async_copy(gbs[u], agg_sh.at[dst_v.at[0]],
                                  sss[u]).wait()
            if counted:
                pltpu.make_async_copy(
                    ones_v, cnt_sh.at[dst_v.at[0]], sss[u]).wait()

        for u in range(3):
            fire_gather(u, u)

        def body(i, carry):
            j0 = 4 * i
            for u in range(4):
                j = j0 + u
                counted = (u % 2) == count_parity
                drain_gather(u)
                fire_scatter(j, u, counted)
                @pl.when(j + 3 < NCHUNK)
                def _():
                    v = (u + 3) % 4

                    @pl.when(j >= 1)
                    def _():
                        drain_scatter(v, ((u + 3) % 2) == count_parity)

                    fire_gather(j + 3, v)
            return carry

        lax.fori_loop(0, NQUAD, body, 0)
        for u in range(4):
            drain_scatter(u, (u % 2) == count_parity)

    @pl.when(c == 0)
    def _():
        run(x0b, 0)

    @pl.when(c == 1)
    def _():
        run(x1b, 1)

    plsc.subcore_barrier()

    pltpu.sync_copy(agg_sh.at[rows], agg_out.at[c, rows])
    pltpu.sync_copy(cnt_sh.at[rows], cnt_out.at[c, rows])


def _sc_aggregate(x0b, x1b, srcs, dsts, zc, ones_h):
    mesh = plsc.VectorSubcoreMesh(core_axis_name="c", subcore_axis_name="s",
                                  num_cores=NCORE, num_subcores=NSUB)
    return pl.kernel(
        _sc_body,
        out_type=(jax.ShapeDtypeStruct((NCORE, NPAD, DH), jnp.bfloat16),
                  jax.ShapeDtypeStruct((NCORE, NPAD, CW), jnp.float32)),
        mesh=mesh,
        scratch_types=[
            pltpu.VMEM_SHARED((NPAD, DH), jnp.bfloat16),
            pltpu.VMEM_SHARED((NPAD, CW), jnp.float32),
            pltpu.VMEM((NCHUNK, CHUNK), jnp.int32),
            pltpu.VMEM((NCHUNK, CHUNK), jnp.int32),
            pltpu.VMEM((CHUNK, DH), jnp.bfloat16),
            pltpu.VMEM((CHUNK, DH), jnp.bfloat16),
            pltpu.VMEM((CHUNK, DH), jnp.bfloat16),
            pltpu.VMEM((CHUNK, DH), jnp.bfloat16),
            pltpu.VMEM((CHUNK, CW), jnp.float32),
            pltpu.SemaphoreType.DMA,
            pltpu.SemaphoreType.DMA,
            pltpu.SemaphoreType.DMA,
            pltpu.SemaphoreType.DMA,
            pltpu.SemaphoreType.DMA,
            pltpu.SemaphoreType.DMA,
            pltpu.SemaphoreType.DMA,
            pltpu.SemaphoreType.DMA,
        ],
        compiler_params=pltpu.CompilerParams(use_tc_tiling_on_sc=False),
    )(x0b, x1b, srcs, dsts, zc, ones_h)


def _tc_body(agg_ref, cnt_ref, w_ref, b_ref, out_ref):
    a = jnp.concatenate([agg_ref[0], agg_ref[1]], axis=1).astype(jnp.float32)
    cnt = cnt_ref[0, :, 0:1] + cnt_ref[1, :, 0:1]
    denom = 1.0 + jnp.maximum(cnt, 1.0)
    a = a / denom
    o = lax.dot_general(a, w_ref[:], (((1,), (1,)), ((), ())),
                        preferred_element_type=jnp.float32)
    o = o + b_ref[:]
    out_ref[:] = jnp.where(o >= 0.0, o, 0.2 * o)


def _tc_tail(agg, cnt, w, b2):
    bm = 512
    return pl.pallas_call(
        _tc_body,
        grid=(pl.cdiv(N, bm),),
        in_specs=[
            pl.BlockSpec((NCORE, bm, DH), lambda i: (0, i, 0)),
            pl.BlockSpec((NCORE, bm, CW), lambda i: (0, i, 0)),
            pl.BlockSpec((DIN, DIN), lambda i: (0, 0)),
            pl.BlockSpec((1, DIN), lambda i: (0, 0)),
        ],
        out_specs=pl.BlockSpec((bm, DIN), lambda i: (i, 0)),
        out_shape=jax.ShapeDtypeStruct((N, DIN), jnp.float32),
    )(agg, cnt, w, b2)


def kernel(x, edge_index, W, b):
    src = edge_index[0].astype(jnp.int32)
    dst = edge_index[1].astype(jnp.int32)
    pad = EPAD - E
    src = jnp.concatenate([src, jnp.arange(pad, dtype=jnp.int32) % N])
    dst = jnp.concatenate(
        [dst, N + (jnp.arange(pad, dtype=jnp.int32) % (NPAD - N))])
    srcs = src.reshape(NSUB, NCHUNK, CHUNK)
    dsts = dst.reshape(NSUB, NCHUNK, CHUNK)
    xp = jnp.pad(x, ((0, NPAD - N), (0, 0)))
    x0b = xp[:, :DH].astype(jnp.bfloat16)
    x1b = xp[:, DH:].astype(jnp.bfloat16)
    zc = jnp.zeros((NPAD, CW), jnp.float32)
    ones_h = jnp.ones((CHUNK, CW), jnp.float32)
    agg, cnt = _sc_aggregate(x0b, x1b, srcs, dsts, zc, ones_h)
    return _tc_tail(agg, cnt, W, b.reshape(1, DIN))


# speedup vs baseline: 2.6250x; 1.0164x over previous
import jax
import jax.numpy as jnp
from jax import lax
from jax.experimental import pallas as pl
from jax.experimental.pallas import tpu as pltpu
from jax.experimental.pallas import tpu_sc as plsc

N = 10000
DIN = 256
DH = 128
E = 160000
NSUB = 16
NCORE = 2
CHUNK = 128
NCHUNK = 79
NQUAD = NCHUNK // 4
EPAD = NSUB * NCHUNK * CHUNK
NPAD = 10112
RPS = NPAD // NSUB
SEED_LAST = N - 15 * RPS
CW = 16


def _sc_body(x0b, x1b, srcs, dsts, zc, ones_h, agg_out, cnt_out,
             agg_sh, cnt_sh, src_v, dst_v, g0, g1, g2, g3, ones_v,
             sg0, sg1, sg2, sg3, ss0, ss1, ss2, ss3):
    c = lax.axis_index("c")
    s = lax.axis_index("s")
    rows = pl.ds(s * RPS, RPS)
    gbs = (g0, g1, g2, g3)
    sgs = (sg0, sg1, sg2, sg3)
    sss = (ss0, ss1, ss2, ss3)

    pltpu.sync_copy(srcs.at[s], src_v)
    pltpu.sync_copy(dsts.at[s], dst_v)
    pltpu.sync_copy(zc.at[rows], cnt_sh.at[rows])
    pltpu.sync_copy(ones_h, ones_v)

    def seed(table):
        @pl.when(s < NSUB - 1)
        def _():
            pltpu.sync_copy(table.at[rows], agg_sh.at[rows])

        @pl.when(s == NSUB - 1)
        def _():
            last = pl.ds((NSUB - 1) * RPS, SEED_LAST)
            pltpu.sync_copy(table.at[last], agg_sh.at[last])

    @pl.when(c == 0)
    def _():
        seed(x0b)

    @pl.when(c == 1)
    def _():
        seed(x1b)

    plsc.subcore_barrier()

    def run(table, count_parity):
        def fire_gather(j, u):
            pltpu.async_copy(table.at[src_v.at[j]], gbs[u], sgs[u])

        def drain_gather(u):
            pltpu.make_async_copy(table.at[src_v.at[0]], gbs[u],
                                  sgs[u]).wait()

        def fire_scatter(j, u, counted):
            pltpu.async_copy(gbs[u], agg_sh.at[dst_v.at[j]], sss[u],
                             add=True)
            if counted:
                pltpu.async_copy(ones_v, cnt_sh.at[dst_v.at[j]], sss[u],
                                 add=True)

        def drain_scatter(u, counted):
            pltpu.make_async_copy(gbs[u], agg_sh.at[dst_v.at[0]],
                                  sss[u]).wait()
            if counted:
                pltpu.make_async_copy(
                    ones_v, cnt_sh.at[dst_v.at[0]], sss[u]).wait()

        for u in range(3):
            fire_gather(u, u)

        def body(i, carry):
            j0 = 4 * i
            for u in range(4):
                j = j0 + u
                counted = (u % 2) == count_parity
                drain_gather(u)
                fire_scatter(j, u, counted)
                @pl.when(j + 3 < NCHUNK)
                def _():
                    v = (u + 3) % 4

                    @pl.when(j >= 1)
                    def _():
                        drain_scatter(v, ((u + 3) % 2) == count_parity)

                    fire_gather(j + 3, v)
            return carry

        lax.fori_loop(0, NQUAD, body, 0)

        for j in range(NQUAD * 4, NCHUNK):
            u = j % 4
            drain_gather(u)
            fire_scatter(j, u, (u % 2) == count_parity)
        for j in range(NCHUNK - 4, NCHUNK):
            u = j % 4
            drain_scatter(u, (u % 2) == count_parity)

    @pl.when(c == 0)
    def _():
        run(x0b, 0)

    @pl.when(c == 1)
    def _():
        run(x1b, 1)

    plsc.subcore_barrier()

    pltpu.sync_copy(agg_sh.at[rows], agg_out.at[c, rows])
    pltpu.sync_copy(cnt_sh.at[rows], cnt_out.at[c, rows])


def _sc_aggregate(x0b, x1b, srcs, dsts, zc, ones_h):
    mesh = plsc.VectorSubcoreMesh(core_axis_name="c", subcore_axis_name="s",
                                  num_cores=NCORE, num_subcores=NSUB)
    return pl.kernel(
        _sc_body,
        out_type=(jax.ShapeDtypeStruct((NCORE, NPAD, DH), jnp.bfloat16),
                  jax.ShapeDtypeStruct((NCORE, NPAD, CW), jnp.float32)),
        mesh=mesh,
        scratch_types=[
            pltpu.VMEM_SHARED((NPAD, DH), jnp.bfloat16),
            pltpu.VMEM_SHARED((NPAD, CW), jnp.float32),
            pltpu.VMEM((NCHUNK, CHUNK), jnp.int32),
            pltpu.VMEM((NCHUNK, CHUNK), jnp.int32),
            pltpu.VMEM((CHUNK, DH), jnp.bfloat16),
            pltpu.VMEM((CHUNK, DH), jnp.bfloat16),
            pltpu.VMEM((CHUNK, DH), jnp.bfloat16),
            pltpu.VMEM((CHUNK, DH), jnp.bfloat16),
            pltpu.VMEM((CHUNK, CW), jnp.float32),
            pltpu.SemaphoreType.DMA,
            pltpu.SemaphoreType.DMA,
            pltpu.SemaphoreType.DMA,
            pltpu.SemaphoreType.DMA,
            pltpu.SemaphoreType.DMA,
            pltpu.SemaphoreType.DMA,
            pltpu.SemaphoreType.DMA,
            pltpu.SemaphoreType.DMA,
        ],
        compiler_params=pltpu.CompilerParams(use_tc_tiling_on_sc=False),
    )(x0b, x1b, srcs, dsts, zc, ones_h)


def _tc_body(agg_ref, cnt_ref, w_ref, b_ref, out_ref):
    a = jnp.concatenate([agg_ref[0], agg_ref[1]], axis=1).astype(jnp.float32)
    cnt = cnt_ref[0, :, 0:1] + cnt_ref[1, :, 0:1]
    denom = 1.0 + jnp.maximum(cnt, 1.0)
    a = (a / denom).astype(jnp.bfloat16)
    w = w_ref[:].astype(jnp.bfloat16)
    o = lax.dot_general(a, w, (((1,), (1,)), ((), ())),
                        preferred_element_type=jnp.float32)
    o = o + b_ref[:]
    out_ref[:] = jnp.where(o >= 0.0, o, 0.2 * o)


def _tc_tail(agg, cnt, w, b2):
    bm = 512
    return pl.pallas_call(
        _tc_body,
        grid=(pl.cdiv(N, bm),),
        in_specs=[
            pl.BlockSpec((NCORE, bm, DH), lambda i: (0, i, 0)),
            pl.BlockSpec((NCORE, bm, CW), lambda i: (0, i, 0)),
            pl.BlockSpec((DIN, DIN), lambda i: (0, 0)),
            pl.BlockSpec((1, DIN), lambda i: (0, 0)),
        ],
        out_specs=pl.BlockSpec((bm, DIN), lambda i: (i, 0)),
        out_shape=jax.ShapeDtypeStruct((N, DIN), jnp.float32),
    )(agg, cnt, w, b2)


def kernel(x, edge_index, W, b):
    src = edge_index[0].astype(jnp.int32)
    dst = edge_index[1].astype(jnp.int32)
    pad = EPAD - E
    src = jnp.concatenate([src, jnp.arange(pad, dtype=jnp.int32) % N])
    dst = jnp.concatenate(
        [dst, N + (jnp.arange(pad, dtype=jnp.int32) % (NPAD - N))])
    srcs = src.reshape(NSUB, NCHUNK, CHUNK)
    dsts = dst.reshape(NSUB, NCHUNK, CHUNK)
    xb = x.astype(jnp.bfloat16)
    x0b = xb[:, :DH]
    x1b = xb[:, DH:]
    zc = jnp.zeros((NPAD, CW), jnp.float32)
    ones_h = jnp.ones((CHUNK, CW), jnp.float32)
    agg, cnt = _sc_aggregate(x0b, x1b, srcs, dsts, zc, ones_h)
    return _tc_tail(agg, cnt, W, b.reshape(1, DIN))
